# Initial kernel scaffold; baseline (speedup 1.0000x reference)
import jax
import jax.numpy as jnp
from jax.experimental import pallas as pl


def kernel(x, w_gate, b_gate, w_noise, b_noise):
    raise NotImplementedError("write your pallas kernel here")


# speedup vs baseline: 3.6277x; 3.6277x over previous
import jax
import jax.numpy as jnp
from jax.experimental import pallas as pl
from jax.experimental.pallas import tpu as pltpu

_INPUT_DIM = 4096
_NUM_EXPERTS = 64
_TOP_K = 8
_NOISE_EPS = 0.01
_LOSS_COEF = 0.01
_TOKENS = 8192

_TB = 512
_NBLK = _TOKENS // _TB


def _gating_kernel(x_ref, w_ref, b_ref, noise_ref, gates_ref, stats_ref, loss_ref):
    i = pl.program_id(0)
    E = _NUM_EXPERTS

    logits2 = (
        jnp.dot(x_ref[...], w_ref[...], preferred_element_type=jnp.float32)
        + b_ref[...]
    )
    clean = logits2[:, :E]
    raw = logits2[:, E:]
    stddev = jnp.logaddexp(raw, 0.0) + _NOISE_EPS
    logits = clean + noise_ref[...] * stddev

    iota = jax.lax.broadcasted_iota(jnp.int32, logits.shape, 1)
    top1 = jnp.max(logits, axis=1, keepdims=True)
    work = logits
    mask = jnp.zeros(logits.shape, jnp.bool_)
    for _ in range(_TOP_K):
        m = jnp.max(work, axis=1, keepdims=True)
        sel_idx = jnp.min(jnp.where(work == m, iota, E), axis=1, keepdims=True)
        sel = iota == sel_idx
        mask = jnp.logical_or(mask, sel)
        work = jnp.where(sel, -jnp.inf, work)

    ex = jnp.where(mask, jnp.exp(logits - top1), 0.0)
    gates = ex / jnp.sum(ex, axis=1, keepdims=True)
    gates_ref[...] = gates

    imp = jnp.sum(gates, axis=0, keepdims=True)
    load = jnp.sum((gates > 0).astype(jnp.float32), axis=0, keepdims=True)

    @pl.when(i == 0)
    def _():
        stats_ref[...] = jnp.zeros_like(stats_ref)

    stats_ref[0:1, :] += imp
    stats_ref[1:2, :] += load

    @pl.when(i == _NBLK - 1)
    def _():
        stats = stats_ref[...]
        n = jnp.float32(E)
        mean = jnp.sum(stats, axis=1, keepdims=True) / n
        var = jnp.sum((stats - mean) ** 2, axis=1, keepdims=True) / (n - 1.0)
        cv2 = var / (mean**2 + 1e-10)
        loss_ref[...] = (cv2[0:1, :] + cv2[1:2, :]) * _LOSS_COEF


def kernel(x, w_gate, b_gate, w_noise, b_noise):
    T, D = x.shape
    E = w_gate.shape[0]
    w = jnp.concatenate([w_gate, w_noise], axis=0).T
    b = jnp.concatenate([b_gate, b_noise])[None, :]
    noise = jax.random.normal(jax.random.key(42), (T, E), dtype=jnp.float32)

    gates, _, loss = pl.pallas_call(
        _gating_kernel,
        grid=(_NBLK,),
        in_specs=[
            pl.BlockSpec((_TB, D), lambda i: (i, 0)),
            pl.BlockSpec((D, 2 * E), lambda i: (0, 0)),
            pl.BlockSpec((1, 2 * E), lambda i: (0, 0)),
            pl.BlockSpec((_TB, E), lambda i: (i, 0)),
        ],
        out_specs=[
            pl.BlockSpec((_TB, E), lambda i: (i, 0)),
            pl.BlockSpec((2, E), lambda i: (0, 0)),
            pl.BlockSpec((1, 1), lambda i: (0, 0)),
        ],
        out_shape=[
            jax.ShapeDtypeStruct((T, E), jnp.float32),
            jax.ShapeDtypeStruct((2, E), jnp.float32),
            jax.ShapeDtypeStruct((1, 1), jnp.float32),
        ],
    )(x, w, b, noise)
    return gates, jnp.reshape(loss, ())


# speedup vs baseline: 5.8128x; 1.6023x over previous
import jax
import jax.numpy as jnp
from jax.experimental import pallas as pl
from jax.experimental.pallas import tpu as pltpu

_INPUT_DIM = 4096
_NUM_EXPERTS = 64
_TOP_K = 8
_NOISE_EPS = 0.01
_LOSS_COEF = 0.01
_TOKENS = 8192

_TB = 512
_NBLK = _TOKENS // _TB


def _gating_kernel(x_ref, w_ref, b_ref, noise_ref, gates_ref, stats_ref, loss_ref):
    i = pl.program_id(0)
    E = _NUM_EXPERTS

    logits2 = (
        jax.lax.dot_general(
            w_ref[...], x_ref[...], (((1,), (1,)), ((), ())),
            preferred_element_type=jnp.float32,
        )
        + b_ref[...]
    )
    clean = logits2[:E, :]
    raw = logits2[E:, :]
    stddev = jnp.logaddexp(raw, 0.0) + _NOISE_EPS
    logits = clean + noise_ref[...] * stddev

    iota = jax.lax.broadcasted_iota(jnp.int32, logits.shape, 0).astype(jnp.float32)
    top1 = jnp.max(logits, axis=0, keepdims=True)
    work = logits
    neg_inf = jnp.float32(-jnp.inf)
    for _ in range(_TOP_K):
        m = jnp.max(work, axis=0, keepdims=True)
        sel_idx = jnp.min(
            jnp.where(work == m, iota, jnp.float32(E)), axis=0, keepdims=True
        )
        work = jnp.where(iota == sel_idx, neg_inf, work)

    mask = work == neg_inf
    ex = jnp.where(mask, jnp.exp(logits - top1), 0.0)
    gates = ex / jnp.sum(ex, axis=0, keepdims=True)
    gates_ref[...] = gates

    imp = jnp.sum(gates, axis=1, keepdims=True)
    load = jnp.sum((gates > 0).astype(jnp.float32), axis=1, keepdims=True)

    @pl.when(i == 0)
    def _():
        stats_ref[...] = jnp.zeros_like(stats_ref)

    stats_ref[:, 0:1] += imp
    stats_ref[:, 1:2] += load

    @pl.when(i == _NBLK - 1)
    def _():
        stats = stats_ref[...]
        n = jnp.float32(E)
        mean = jnp.sum(stats, axis=0, keepdims=True) / n
        var = jnp.sum((stats - mean) ** 2, axis=0, keepdims=True) / (n - 1.0)
        cv2 = var / (mean**2 + 1e-10)
        loss_ref[...] = (cv2[:, 0:1] + cv2[:, 1:2]) * _LOSS_COEF


def kernel(x, w_gate, b_gate, w_noise, b_noise):
    T, D = x.shape
    E = w_gate.shape[0]
    w = jnp.concatenate([w_gate, w_noise], axis=0)
    b = jnp.concatenate([b_gate, b_noise])[:, None]
    noise_t = jax.random.normal(jax.random.key(42), (T, E), dtype=jnp.float32).T

    gates_t, _, loss = pl.pallas_call(
        _gating_kernel,
        grid=(_NBLK,),
        in_specs=[
            pl.BlockSpec((_TB, D), lambda i: (i, 0)),
            pl.BlockSpec((2 * E, D), lambda i: (0, 0)),
            pl.BlockSpec((2 * E, 1), lambda i: (0, 0)),
            pl.BlockSpec((E, _TB), lambda i: (0, i)),
        ],
        out_specs=[
            pl.BlockSpec((E, _TB), lambda i: (0, i)),
            pl.BlockSpec((E, 2), lambda i: (0, 0)),
            pl.BlockSpec((1, 1), lambda i: (0, 0)),
        ],
        out_shape=[
            jax.ShapeDtypeStruct((E, T), jnp.float32),
            jax.ShapeDtypeStruct((E, 2), jnp.float32),
            jax.ShapeDtypeStruct((1, 1), jnp.float32),
        ],
    )(x, w, b, noise_t)
    return gates_t.T, jnp.reshape(loss, ())


# speedup vs baseline: 6.1255x; 1.0538x over previous
import jax
import jax.numpy as jnp
from jax.experimental import pallas as pl
from jax.experimental.pallas import tpu as pltpu

_INPUT_DIM = 4096
_NUM_EXPERTS = 64
_TOP_K = 8
_NOISE_EPS = 0.01
_LOSS_COEF = 0.01
_TOKENS = 8192

_TB = 1024
_NBLK = _TOKENS // _TB


def _gating_kernel(x_ref, w_ref, b_ref, noise_ref, gates_ref, stats_ref, loss_ref):
    i = pl.program_id(0)
    E = _NUM_EXPERTS

    logits2 = (
        jax.lax.dot_general(
            w_ref[...], x_ref[...], (((1,), (1,)), ((), ())),
            preferred_element_type=jnp.float32,
        )
        + b_ref[...]
    )
    clean = logits2[:E, :]
    raw = logits2[E:, :]
    stddev = jnp.logaddexp(raw, 0.0) + _NOISE_EPS
    logits = clean + noise_ref[...] * stddev

    iota = jax.lax.broadcasted_iota(jnp.int32, logits.shape, 0).astype(jnp.float32)
    top1 = jnp.max(logits, axis=0, keepdims=True)
    work = logits
    neg_inf = jnp.float32(-jnp.inf)
    for _ in range(_TOP_K):
        m = jnp.max(work, axis=0, keepdims=True)
        sel_idx = jnp.min(
            jnp.where(work == m, iota, jnp.float32(E)), axis=0, keepdims=True
        )
        work = jnp.where(iota == sel_idx, neg_inf, work)

    mask = work == neg_inf
    ex = jnp.where(mask, jnp.exp(logits - top1), 0.0)
    gates = ex / jnp.sum(ex, axis=0, keepdims=True)
    gates_ref[...] = gates

    imp = jnp.sum(gates, axis=1, keepdims=True)
    load = jnp.sum((gates > 0).astype(jnp.float32), axis=1, keepdims=True)

    @pl.when(i == 0)
    def _():
        stats_ref[...] = jnp.zeros_like(stats_ref)

    stats_ref[:, 0:1] += imp
    stats_ref[:, 1:2] += load

    @pl.when(i == _NBLK - 1)
    def _():
        stats = stats_ref[...]
        n = jnp.float32(E)
        mean = jnp.sum(stats, axis=0, keepdims=True) / n
        var = jnp.sum((stats - mean) ** 2, axis=0, keepdims=True) / (n - 1.0)
        cv2 = var / (mean**2 + 1e-10)
        loss_ref[...] = (cv2[:, 0:1] + cv2[:, 1:2]) * _LOSS_COEF


def kernel(x, w_gate, b_gate, w_noise, b_noise):
    T, D = x.shape
    E = w_gate.shape[0]
    w = jnp.concatenate([w_gate, w_noise], axis=0)
    b = jnp.concatenate([b_gate, b_noise])[:, None]
    noise_t = jax.random.normal(jax.random.key(42), (T, E), dtype=jnp.float32).T

    gates_t, _, loss = pl.pallas_call(
        _gating_kernel,
        grid=(_NBLK,),
        in_specs=[
            pl.BlockSpec((_TB, D), lambda i: (i, 0)),
            pl.BlockSpec((2 * E, D), lambda i: (0, 0)),
            pl.BlockSpec((2 * E, 1), lambda i: (0, 0)),
            pl.BlockSpec((E, _TB), lambda i: (0, i)),
        ],
        out_specs=[
            pl.BlockSpec((E, _TB), lambda i: (0, i)),
            pl.BlockSpec((E, 2), lambda i: (0, 0)),
            pl.BlockSpec((1, 1), lambda i: (0, 0)),
        ],
        out_shape=[
            jax.ShapeDtypeStruct((E, T), jnp.float32),
            jax.ShapeDtypeStruct((E, 2), jnp.float32),
            jax.ShapeDtypeStruct((1, 1), jnp.float32),
        ],
    )(x, w, b, noise_t)
    return gates_t.T, jnp.reshape(loss, ())
